# baseline (speedup 1.0000x reference)
import functools

import jax
import jax.numpy as jnp
from jax import lax
from jax.experimental import pallas as pl
from jax.experimental.pallas import tpu as pltpu
from jax.experimental.pallas import tpu_sc as plsc

K = 20
NUM_GRAPHS = 8
R = 256
RK = 1024
TC_COL = 256


KPAD = 24


def _knn_body(starts_ref, ends_ref, wlo_ref, whi_ref,
              xr_ref, xf_ref, brow_ref, idx_ref, d2_ref):
    t = pl.program_id(0)
    d = xr_ref.shape[1]
    xr = xr_ref[...]
    dn = (((1,), (1,)), ((), ()))

    brow = brow_ref[...]
    st = jnp.zeros((1, RK), jnp.int32)
    en = jnp.zeros((1, RK), jnp.int32)
    for g in range(NUM_GRAPHS):
        st = jnp.where(brow == g, starts_ref[g], st)
        en = jnp.where(brow == g, ends_ref[g], en)
    rid = t * RK + lax.broadcasted_iota(jnp.int32, (1, RK), 1)
    scol = lax.broadcasted_iota(jnp.int32, (TC_COL, 1), 0)
    ones = jnp.ones((1, d), jnp.float32)
    rs = lax.dot_general(ones, xr * xr, dn,
                         precision=lax.Precision.HIGHEST,
                         preferred_element_type=jnp.float32)
    lo = wlo_ref[t]
    hi = whi_ref[t]

    def fill(c, _):
        xc = xf_ref[pl.ds(c * TC_COL, TC_COL), :]
        cs = jnp.sum(xc * xc, axis=1, keepdims=True)
        dot = lax.dot_general(xc, xr, dn,
                              preferred_element_type=jnp.float32)
        d2 = rs + cs - 2.0 * dot
        cid = c * TC_COL + scol
        same = (cid >= st) & (cid < en)
        d2 = jnp.where(same, d2, jnp.float32(1e9))
        d2 = d2 + jnp.where(cid == rid, jnp.float32(1e9), jnp.float32(0.0))
        d2_ref[c] = d2
        return 0

    lax.fori_loop(lo, hi, fill, 0)

    scol_f = scol.astype(jnp.float32)
    bigf = jnp.float32(2**30)
    a_prev = jnp.full((1, RK), jnp.float32(-1.0))
    for r in range(K):
        def scan(c, carry):
            m, a, ap = carry
            blk = d2_ref[c]
            cid = jnp.float32(c * TC_COL) + scol_f
            blk = jnp.where(cid == ap, jnp.float32(2e9), blk)
            d2_ref[c] = blk
            tmin = jnp.min(blk, axis=0, keepdims=True)
            cand = jnp.where(blk == tmin, jnp.broadcast_to(cid, blk.shape),
                             bigf)
            targ = jnp.min(cand, axis=0, keepdims=True)
            take = tmin < m
            return (jnp.where(take, tmin, m), jnp.where(take, targ, a), ap)

        m0 = jnp.full((1, RK), jnp.float32(3e9))
        a0 = jnp.full((1, RK), bigf)
        _, a, _ = lax.fori_loop(lo, hi, scan, (m0, a0, a_prev))
        idx_ref[r:r + 1, :] = a.astype(jnp.int32)
        a_prev = a


def _knn(x, brow1n, starts, ends, wlo, whi, interpret=False):
    n, d = x.shape
    nt = n // RK
    return pl.pallas_call(
        _knn_body,
        grid=(nt,),
        in_specs=[
            pl.BlockSpec(memory_space=pltpu.SMEM),
            pl.BlockSpec(memory_space=pltpu.SMEM),
            pl.BlockSpec(memory_space=pltpu.SMEM),
            pl.BlockSpec(memory_space=pltpu.SMEM),
            pl.BlockSpec((RK, d), lambda t: (t, 0)),
            pl.BlockSpec((n, d), lambda t: (0, 0)),
            pl.BlockSpec((1, RK), lambda t: (0, t)),
        ],
        out_specs=pl.BlockSpec((KPAD, RK), lambda t: (0, t)),
        out_shape=jax.ShapeDtypeStruct((KPAD, n), jnp.int32),
        scratch_shapes=[pltpu.VMEM((n // TC_COL, TC_COL, RK), jnp.float32)],
        interpret=interpret,
    )(starts, ends, wlo, whi, x, x, brow1n)


def _sc_gather(x, idxt_flat):
    n, d = x.shape
    tot = idxt_flat.shape[0]
    nw = 32
    per_w = tot // nw
    ch = 128
    nch = per_w // ch

    mesh = plsc.VectorSubcoreMesh(core_axis_name="c", subcore_axis_name="s")

    @functools.partial(
        pl.kernel,
        mesh=mesh,
        out_type=jax.ShapeDtypeStruct((tot, d), jnp.float32),
        scratch_types=[
            pltpu.VMEM((ch,), jnp.int32),
            pltpu.VMEM((ch, d), jnp.float32),
            pltpu.SemaphoreType.DMA,
        ],
    )
    def body(x_hbm, idx_hbm, out_hbm, idx_v, rows_v, sem):
        wid = lax.axis_index("s") * 2 + lax.axis_index("c")
        base = wid * per_w

        def chunk(i, _):
            rb = base + i * ch
            pltpu.sync_copy(idx_hbm.at[pl.ds(rb, ch)], idx_v)
            pltpu.async_copy(x_hbm.at[idx_v], rows_v, sem).wait()
            pltpu.sync_copy(rows_v, out_hbm.at[pl.ds(rb, ch), :])
            return 0

        lax.fori_loop(0, nch, chunk, 0)

    return body(x, idxt_flat)


def _edge_mlp_body(xi_ref, xg_ref, w_ref, b_ref, h_ref):
    xi = xi_ref[...]
    dn = (((1,), (1,)), ((), ()))
    w = w_ref[...]
    b = b_ref[...]
    acc = None
    for t in range(K):
        xj = xg_ref[t]
        e = jnp.concatenate([xi, xj - xi], axis=1)
        ht = lax.dot_general(e, w, dn, preferred_element_type=jnp.float32)
        ht = jnp.maximum(ht + b, 0.0)
        acc = ht if acc is None else jnp.maximum(acc, ht)
    h_ref[...] = acc


def _edge_mlp_max(x, xg, w, b, interpret=False):
    n, d = x.shape
    dout = w.shape[0]
    nt = n // R
    return pl.pallas_call(
        _edge_mlp_body,
        grid=(nt,),
        in_specs=[
            pl.BlockSpec((R, d), lambda t: (t, 0)),
            pl.BlockSpec((K, R, d), lambda t: (0, t, 0)),
            pl.BlockSpec((dout, 2 * d), lambda t: (0, 0)),
            pl.BlockSpec((1, dout), lambda t: (0, 0)),
        ],
        out_specs=pl.BlockSpec((R, dout), lambda t: (t, 0)),
        out_shape=jax.ShapeDtypeStruct((n, dout), jnp.float32),
        interpret=interpret,
    )(x, xg, w, b)


def _pool_fc_body(h_ref, brow_ref, wfc_ref, bfc_ref, out_ref, pool_ref):
    t = pl.program_id(0)
    nt = pl.num_programs(0)
    c = h_ref.shape[1]

    @pl.when(t == 0)
    def _():
        pool_ref[...] = jnp.full((NUM_GRAPHS, c), jnp.float32(-1e30))

    h = h_ref[...]
    brow = brow_ref[...]
    for g in range(NUM_GRAPHS):
        m = jnp.max(jnp.where(brow == g, h, jnp.float32(-1e30)),
                    axis=0, keepdims=True)
        pool_ref[g:g + 1, :] = jnp.maximum(pool_ref[g:g + 1, :], m)

    @pl.when(t == nt - 1)
    def _():
        dn = (((1,), (1,)), ((), ()))
        out = lax.dot_general(pool_ref[...], wfc_ref[...], dn,
                              preferred_element_type=jnp.float32)
        out_ref[...] = jnp.maximum(out + bfc_ref[...], 0.0)


def _pool_fc(h, brow, wfc, bfc, interpret=False):
    n, c = h.shape
    nt = n // R
    dout = wfc.shape[0]
    return pl.pallas_call(
        _pool_fc_body,
        grid=(nt,),
        in_specs=[
            pl.BlockSpec((R, c), lambda t: (t, 0)),
            pl.BlockSpec((R, 1), lambda t: (t, 0)),
            pl.BlockSpec((dout, c), lambda t: (0, 0)),
            pl.BlockSpec((1, dout), lambda t: (0, 0)),
        ],
        out_specs=pl.BlockSpec((NUM_GRAPHS, dout), lambda t: (0, 0)),
        out_shape=jax.ShapeDtypeStruct((NUM_GRAPHS, dout), jnp.float32),
        scratch_shapes=[pltpu.VMEM((NUM_GRAPHS, c), jnp.float32)],
        interpret=interpret,
    )(h, brow, wfc, bfc)


def kernel(x, batch, W1, b1, W2, b2, W3, b3, Wfc, bfc):
    n = x.shape[0]
    bi = batch.astype(jnp.int32)
    gids = jnp.arange(NUM_GRAPHS, dtype=jnp.int32)
    starts = jnp.searchsorted(bi, gids, side="left").astype(jnp.int32)
    ends = jnp.searchsorted(bi, gids, side="right").astype(jnp.int32)
    g_first = bi[0::RK]
    g_last = bi[RK - 1::RK]
    wlo = (starts[g_first] // TC_COL).astype(jnp.int32)
    whi = ((ends[g_last] + TC_COL - 1) // TC_COL).astype(jnp.int32)
    brow = bi.reshape(n, 1)
    brow1n = bi.reshape(1, n)

    h = x
    for w, b in ((W1, b1), (W2, b2), (W3, b3)):
        d = h.shape[1]
        idx = _knn(h, brow1n, starts, ends, wlo, whi)
        idxt = idx[:K].reshape(-1)
        xg = _sc_gather(h, idxt)
        h = _edge_mlp_max(h, xg.reshape(K, n, d), w, b.reshape(1, -1))

    return _pool_fc(h, brow, Wfc, bfc.reshape(1, -1))


# speedup vs baseline: 1.1552x; 1.1552x over previous
import functools

import jax
import jax.numpy as jnp
from jax import lax
from jax.experimental import pallas as pl
from jax.experimental.pallas import tpu as pltpu
from jax.experimental.pallas import tpu_sc as plsc

K = 20
NUM_GRAPHS = 8
R = 256
RK = 512
TC_COL = 256


KPAD = 24


def _knn_body(starts_ref, ends_ref, wlo_ref, whi_ref,
              xr_ref, xf_ref, brow_ref, idx_ref, d2_ref):
    t = pl.program_id(0)
    d = xr_ref.shape[1]
    xr = xr_ref[...]
    dn = (((1,), (1,)), ((), ()))

    brow = brow_ref[...]
    st = jnp.zeros((1, RK), jnp.int32)
    en = jnp.zeros((1, RK), jnp.int32)
    for g in range(NUM_GRAPHS):
        st = jnp.where(brow == g, starts_ref[g], st)
        en = jnp.where(brow == g, ends_ref[g], en)
    rid = t * RK + lax.broadcasted_iota(jnp.int32, (1, RK), 1)
    scol = lax.broadcasted_iota(jnp.int32, (TC_COL, 1), 0)
    ones = jnp.ones((1, d), jnp.float32)
    rs = lax.dot_general(ones, xr * xr, dn,
                         precision=lax.Precision.HIGHEST,
                         preferred_element_type=jnp.float32)
    lo = wlo_ref[t]
    hi = whi_ref[t]

    def fill(c, _):
        xc = xf_ref[pl.ds(c * TC_COL, TC_COL), :]
        cs = jnp.sum(xc * xc, axis=1, keepdims=True)
        dot = lax.dot_general(xc, xr, dn,
                              preferred_element_type=jnp.float32)
        d2 = rs + cs - 2.0 * dot
        cid = c * TC_COL + scol
        same = (cid >= st) & (cid < en)
        d2 = jnp.where(same, d2, jnp.float32(1e9))
        d2 = d2 + jnp.where(cid == rid, jnp.float32(1e9), jnp.float32(0.0))
        d2_ref[c] = d2
        return 0

    lax.fori_loop(lo, hi, fill, 0)

    scol_f = scol.astype(jnp.float32)
    bigf = jnp.float32(2**30)
    a_prev = jnp.full((1, RK), jnp.float32(-1.0))
    for r in range(K):
        def scan(c, carry):
            m, a, ap = carry
            blk = d2_ref[c]
            cid = jnp.float32(c * TC_COL) + scol_f
            blk = jnp.where(cid == ap, jnp.float32(2e9), blk)
            if r < K - 1:
                d2_ref[c] = blk
            tmin = jnp.min(blk, axis=0, keepdims=True)
            cand = jnp.where(blk == tmin, jnp.broadcast_to(cid, blk.shape),
                             bigf)
            targ = jnp.min(cand, axis=0, keepdims=True)
            take = tmin < m
            return (jnp.where(take, tmin, m), jnp.where(take, targ, a), ap)

        m0 = jnp.full((1, RK), jnp.float32(3e9))
        a0 = jnp.full((1, RK), bigf)
        _, a, _ = lax.fori_loop(lo, hi, scan, (m0, a0, a_prev))
        idx_ref[r:r + 1, :] = a.astype(jnp.int32)
        a_prev = a


def _knn(x, brow1n, starts, ends, wlo, whi, interpret=False):
    n, d = x.shape
    nt = n // RK
    return pl.pallas_call(
        _knn_body,
        grid=(nt,),
        in_specs=[
            pl.BlockSpec(memory_space=pltpu.SMEM),
            pl.BlockSpec(memory_space=pltpu.SMEM),
            pl.BlockSpec(memory_space=pltpu.SMEM),
            pl.BlockSpec(memory_space=pltpu.SMEM),
            pl.BlockSpec((RK, d), lambda t: (t, 0)),
            pl.BlockSpec((n, d), lambda t: (0, 0)),
            pl.BlockSpec((1, RK), lambda t: (0, t)),
        ],
        out_specs=pl.BlockSpec((KPAD, RK), lambda t: (0, t)),
        out_shape=jax.ShapeDtypeStruct((KPAD, n), jnp.int32),
        scratch_shapes=[pltpu.VMEM((n // TC_COL, TC_COL, RK), jnp.float32)],
        interpret=interpret,
    )(starts, ends, wlo, whi, x, x, brow1n)


def _sc_gather(x, idxt_flat):
    n, d = x.shape
    tot = idxt_flat.shape[0]
    nw = 32
    per_w = tot // nw
    ch = 128
    nch = per_w // ch

    mesh = plsc.VectorSubcoreMesh(core_axis_name="c", subcore_axis_name="s")

    assert nch % 2 == 0

    @functools.partial(
        pl.kernel,
        mesh=mesh,
        out_type=jax.ShapeDtypeStruct((tot, d), jnp.float32),
        scratch_types=[
            pltpu.VMEM((ch,), jnp.int32),
            pltpu.VMEM((ch,), jnp.int32),
            pltpu.VMEM((ch, d), jnp.float32),
            pltpu.VMEM((ch, d), jnp.float32),
            pltpu.SemaphoreType.DMA,
            pltpu.SemaphoreType.DMA,
        ],
    )
    def body(x_hbm, idx_hbm, out_hbm, idx_v0, idx_v1, rows_v0, rows_v1,
             sem0, sem1):
        wid = lax.axis_index("s") * 2 + lax.axis_index("c")
        base = wid * per_w

        pltpu.sync_copy(idx_hbm.at[pl.ds(base, ch)], idx_v0)
        pltpu.async_copy(x_hbm.at[idx_v0], rows_v0, sem0)

        def pair(p, _):
            j = 2 * p
            rb0 = base + j * ch
            rb1 = rb0 + ch
            pltpu.sync_copy(idx_hbm.at[pl.ds(rb1, ch)], idx_v1)
            pltpu.async_copy(x_hbm.at[idx_v1], rows_v1, sem1)
            pltpu.make_async_copy(x_hbm.at[idx_v0], rows_v0, sem0).wait()
            pltpu.sync_copy(rows_v0, out_hbm.at[pl.ds(rb0, ch), :])

            @pl.when(j + 2 < nch)
            def _():
                rb2 = rb0 + 2 * ch
                pltpu.sync_copy(idx_hbm.at[pl.ds(rb2, ch)], idx_v0)
                pltpu.async_copy(x_hbm.at[idx_v0], rows_v0, sem0)

            pltpu.make_async_copy(x_hbm.at[idx_v1], rows_v1, sem1).wait()
            pltpu.sync_copy(rows_v1, out_hbm.at[pl.ds(rb1, ch), :])
            return 0

        lax.fori_loop(0, nch // 2, pair, 0)

    return body(x, idxt_flat)


def _edge_mlp_body(xi_ref, xg_ref, w_ref, b_ref, h_ref):
    xi = xi_ref[...]
    dn = (((1,), (1,)), ((), ()))
    w = w_ref[...]
    b = b_ref[...]
    acc = None
    for t in range(K):
        xj = xg_ref[t]
        e = jnp.concatenate([xi, xj - xi], axis=1)
        ht = lax.dot_general(e, w, dn, preferred_element_type=jnp.float32)
        ht = jnp.maximum(ht + b, 0.0)
        acc = ht if acc is None else jnp.maximum(acc, ht)
    h_ref[...] = acc


def _edge_mlp_max(x, xg, w, b, interpret=False):
    n, d = x.shape
    dout = w.shape[0]
    nt = n // R
    return pl.pallas_call(
        _edge_mlp_body,
        grid=(nt,),
        in_specs=[
            pl.BlockSpec((R, d), lambda t: (t, 0)),
            pl.BlockSpec((K, R, d), lambda t: (0, t, 0)),
            pl.BlockSpec((dout, 2 * d), lambda t: (0, 0)),
            pl.BlockSpec((1, dout), lambda t: (0, 0)),
        ],
        out_specs=pl.BlockSpec((R, dout), lambda t: (t, 0)),
        out_shape=jax.ShapeDtypeStruct((n, dout), jnp.float32),
        interpret=interpret,
    )(x, xg, w, b)


def _pool_fc_body(h_ref, brow_ref, wfc_ref, bfc_ref, out_ref, pool_ref):
    t = pl.program_id(0)
    nt = pl.num_programs(0)
    c = h_ref.shape[1]

    @pl.when(t == 0)
    def _():
        pool_ref[...] = jnp.full((NUM_GRAPHS, c), jnp.float32(-1e30))

    h = h_ref[...]
    brow = brow_ref[...]
    for g in range(NUM_GRAPHS):
        m = jnp.max(jnp.where(brow == g, h, jnp.float32(-1e30)),
                    axis=0, keepdims=True)
        pool_ref[g:g + 1, :] = jnp.maximum(pool_ref[g:g + 1, :], m)

    @pl.when(t == nt - 1)
    def _():
        dn = (((1,), (1,)), ((), ()))
        out = lax.dot_general(pool_ref[...], wfc_ref[...], dn,
                              preferred_element_type=jnp.float32)
        out_ref[...] = jnp.maximum(out + bfc_ref[...], 0.0)


def _pool_fc(h, brow, wfc, bfc, interpret=False):
    n, c = h.shape
    nt = n // R
    dout = wfc.shape[0]
    return pl.pallas_call(
        _pool_fc_body,
        grid=(nt,),
        in_specs=[
            pl.BlockSpec((R, c), lambda t: (t, 0)),
            pl.BlockSpec((R, 1), lambda t: (t, 0)),
            pl.BlockSpec((dout, c), lambda t: (0, 0)),
            pl.BlockSpec((1, dout), lambda t: (0, 0)),
        ],
        out_specs=pl.BlockSpec((NUM_GRAPHS, dout), lambda t: (0, 0)),
        out_shape=jax.ShapeDtypeStruct((NUM_GRAPHS, dout), jnp.float32),
        scratch_shapes=[pltpu.VMEM((NUM_GRAPHS, c), jnp.float32)],
        interpret=interpret,
    )(h, brow, wfc, bfc)


def kernel(x, batch, W1, b1, W2, b2, W3, b3, Wfc, bfc):
    n = x.shape[0]
    bi = batch.astype(jnp.int32)
    gids = jnp.arange(NUM_GRAPHS, dtype=jnp.int32)
    starts = jnp.searchsorted(bi, gids, side="left").astype(jnp.int32)
    ends = jnp.searchsorted(bi, gids, side="right").astype(jnp.int32)
    g_first = bi[0::RK]
    g_last = bi[RK - 1::RK]
    wlo = (starts[g_first] // TC_COL).astype(jnp.int32)
    whi = ((ends[g_last] + TC_COL - 1) // TC_COL).astype(jnp.int32)
    brow = bi.reshape(n, 1)
    brow1n = bi.reshape(1, n)

    h = x
    for w, b in ((W1, b1), (W2, b2), (W3, b3)):
        d = h.shape[1]
        idx = _knn(h, brow1n, starts, ends, wlo, whi)
        idxt = idx[:K].reshape(-1)
        xg = _sc_gather(h, idxt)
        h = _edge_mlp_max(h, xg.reshape(K, n, d), w, b.reshape(1, -1))

    return _pool_fc(h, brow, Wfc, bfc.reshape(1, -1))
